# Initial kernel scaffold; baseline (speedup 1.0000x reference)
import jax
import jax.numpy as jnp
from jax.experimental import pallas as pl


def kernel(node_feats, edge_feats, edge_index, Wsg, bsg, Wdg, bdg, Weg, beg, Wxg, bxg, Wmg, bmg, att_src, att_dst, Wsu, bsu, Wdu, bdu, gn, bnb, ge, beb):
    raise NotImplementedError("write your pallas kernel here")


# speedup vs baseline: 2.9413x; 2.9413x over previous
import functools

import jax
import jax.numpy as jnp
from jax import lax
from jax.experimental import pallas as pl
from jax.experimental.pallas import tpu as pltpu
from jax.experimental.pallas import tpu_sc as plsc

N = 10000
E = 320000
D = 128
NBIN = 32
BINW = 313
ACC_ROWS = 320

BN = 2000
BE = 2000

EW = E // 32
CHB = 80
CHC = 80
CHD = 64
ZB = 1248

RECN = E + 384
RTRASH = RECN
BI_LEN = 33 * 16 + 32 * 16


def _node_body(nf_ref, wsg_ref, bsg_ref, wdg_ref, bdg_ref, wdu_ref, bdu_ref,
               wsu_ref, bsu_ref, vs_ref, vd_ref,
               t_ref, atts_ref, attd_ref, axu_ref):
    nf = nf_ref[...]
    e_src = jnp.dot(nf, wsg_ref[...], preferred_element_type=jnp.float32) + bsg_ref[...]
    e_dst = jnp.dot(nf, wdg_ref[...], preferred_element_type=jnp.float32) + bdg_ref[...]
    bh = jnp.dot(nf, wdu_ref[...], preferred_element_type=jnp.float32) + bdu_ref[...]
    axu_ref[...] = jnp.dot(nf, wsu_ref[...], preferred_element_type=jnp.float32) + bsu_ref[...]
    t_ref[:, 0:D] = e_src
    t_ref[:, D:2 * D] = e_dst
    t_ref[:, 2 * D:3 * D] = bh
    atts_ref[...] = jnp.dot(e_src, vs_ref[...], preferred_element_type=jnp.float32)
    attd_ref[...] = jnp.dot(e_dst, vd_ref[...], preferred_element_type=jnp.float32)


def _node_precompute(nf, wsg, bsg, wdg, bdg, wdu, bdu, wsu, bsu, vs, vd):
    wspec = pl.BlockSpec((D, D), lambda i: (0, 0))
    bspec = pl.BlockSpec((1, D), lambda i: (0, 0))
    vspec = pl.BlockSpec((D, 1), lambda i: (0, 0))
    return pl.pallas_call(
        _node_body,
        grid=(N // BN,),
        in_specs=[pl.BlockSpec((BN, D), lambda i: (i, 0)),
                  wspec, bspec, wspec, bspec, wspec, bspec, wspec, bspec,
                  vspec, vspec],
        out_specs=[pl.BlockSpec((BN, 3 * D), lambda i: (i, 0)),
                   pl.BlockSpec((BN, 1), lambda i: (i, 0)),
                   pl.BlockSpec((BN, 1), lambda i: (i, 0)),
                   pl.BlockSpec((BN, D), lambda i: (i, 0))],
        out_shape=[jax.ShapeDtypeStruct((N, 3 * D), jnp.float32),
                   jax.ShapeDtypeStruct((N, 1), jnp.float32),
                   jax.ShapeDtypeStruct((N, 1), jnp.float32),
                   jax.ShapeDtypeStruct((N, D), jnp.float32)],
    )(nf, wsg, bsg, wdg, bdg, wdu, bdu, wsu, bsu, vs, vd)


def _shift_body(as_ref, ad_ref, os_ref, od_ref):
    a_s = as_ref[...]
    a_d = ad_ref[...]
    os_ref[...] = a_s - jnp.max(a_s)
    od_ref[...] = a_d - jnp.max(a_d)


def _shift_att(atts, attd):
    return pl.pallas_call(
        _shift_body,
        out_shape=[jax.ShapeDtypeStruct((N, 1), jnp.float32),
                   jax.ShapeDtypeStruct((N, 1), jnp.float32)],
    )(atts, attd)


def _ef_body(ef_ref, w_ref, b_ref, out_ref):
    out_ref[...] = (jnp.dot(ef_ref[...], w_ref[...],
                            preferred_element_type=jnp.float32) + b_ref[...])


def _edge_matmul(ef, weg, beg):
    return pl.pallas_call(
        _ef_body,
        grid=(E // BE,),
        in_specs=[pl.BlockSpec((BE, D), lambda i: (i, 0)),
                  pl.BlockSpec((D, D), lambda i: (0, 0)),
                  pl.BlockSpec((1, D), lambda i: (0, 0))],
        out_specs=pl.BlockSpec((BE, D), lambda i: (i, 0)),
        out_shape=jax.ShapeDtypeStruct((E, D), jnp.float32),
    )(ef, weg, beg)


def _hist_body(dst_ref, cnt_ref):
    d = dst_ref[...]
    bins = jnp.floor((d.astype(jnp.float32) + 0.5) *
                     (1.0 / BINW)).astype(jnp.int32)
    oh = (bins[0, 0, :, None] ==
          lax.broadcasted_iota(jnp.int32, (EW, NBIN), 1))
    cnt_ref[...] = jnp.sum(oh.astype(jnp.float32), axis=0)[None, None, :]


def _bin_hist(dst32):
    return pl.pallas_call(
        _hist_body,
        grid=(32,),
        in_specs=[pl.BlockSpec((1, 1, EW), lambda i: (i, 0, 0))],
        out_specs=pl.BlockSpec((1, 1, NBIN), lambda i: (i, 0, 0)),
        out_shape=jax.ShapeDtypeStruct((32, 1, NBIN), jnp.float32),
    )(dst32)


def _pos_body(dst_ref, start_ref, pos_ref):
    d = dst_ref[...]
    bins = jnp.floor((d[0, 0, :].astype(jnp.float32) + 0.5) *
                     (1.0 / BINW)).astype(jnp.int32)
    oh = (bins[:, None] ==
          lax.broadcasted_iota(jnp.int32, (EW, NBIN), 1)).astype(jnp.float32)
    rank_excl = jnp.cumsum(oh, axis=0) - oh
    start_row = start_ref[...][0]
    pos = jnp.sum(oh * (rank_excl + start_row), axis=1)
    pos_ref[...] = pos.astype(jnp.int32)[None, None, :]


def _bin_pos(dst32, start3):
    return pl.pallas_call(
        _pos_body,
        grid=(32,),
        in_specs=[pl.BlockSpec((1, 1, EW), lambda i: (i, 0, 0)),
                  pl.BlockSpec((1, 1, NBIN), lambda i: (i, 0, 0))],
        out_specs=pl.BlockSpec((1, 1, EW), lambda i: (i, 0, 0)),
        out_shape=jax.ShapeDtypeStruct((32, 1, EW), jnp.int32),
    )(dst32, start3)


def _dsum_body(dp_ref, dsum_ref):
    dp = dp_ref[...]
    dsum_ref[...] = (dp[0, :] + dp[1, :])[:, None]


def _reduce_denoms(dp):
    return pl.pallas_call(
        _dsum_body,
        out_shape=jax.ShapeDtypeStruct((2 * N, 1), jnp.float32),
    )(dp)


def _bin_plan(cnt):
    tot = jnp.sum(cnt, axis=0).astype(jnp.int32)
    padtot = ((tot + 7) // 8) * 8
    bp33 = jnp.concatenate([jnp.zeros((1,), jnp.int32), jnp.cumsum(padtot)])
    start = bp33[None, :NBIN] + jnp.concatenate(
        [jnp.zeros((1, NBIN), jnp.int32),
         jnp.cumsum(cnt.astype(jnp.int32), axis=0)[:-1]], axis=0)
    binfo = jnp.concatenate([jnp.repeat(bp33, 16), jnp.repeat(tot, 16)])
    return jnp.repeat(start.reshape(32 * NBIN), 16), binfo


def _y_body(m_ref, ef_ref, w_ref, b_ref, g_ref, gb_ref, out_ref):
    t = jnp.dot(m_ref[...], w_ref[...], preferred_element_type=jnp.float32) + b_ref[...]
    mu = jnp.mean(t, axis=1, keepdims=True)
    var = jnp.mean((t - mu) ** 2, axis=1, keepdims=True)
    xn = (t - mu) * lax.rsqrt(var + 1e-5) * g_ref[...] + gb_ref[...]
    out_ref[...] = ef_ref[...] + xn * jax.nn.sigmoid(xn)


def _y_update(m_full, ef, wmg, bmg, ge, beb):
    return pl.pallas_call(
        _y_body,
        grid=(E // BE,),
        in_specs=[pl.BlockSpec((BE, D), lambda i: (i, 0)),
                  pl.BlockSpec((BE, D), lambda i: (i, 0)),
                  pl.BlockSpec((D, D), lambda i: (0, 0)),
                  pl.BlockSpec((1, D), lambda i: (0, 0)),
                  pl.BlockSpec((1, D), lambda i: (0, 0)),
                  pl.BlockSpec((1, D), lambda i: (0, 0))],
        out_specs=pl.BlockSpec((BE, D), lambda i: (i, 0)),
        out_shape=jax.ShapeDtypeStruct((E, D), jnp.float32),
    )(m_full, ef, wmg, bmg, ge, beb)


def _x_body(ssh_ref, ss_ref, axu_ref, nf_ref, w_ref, b_ref, g_ref, gb_ref,
            out_ref):
    h = ssh_ref[...] / (ss_ref[...] + 1e-6)
    x1 = axu_ref[...] + h
    t = jnp.dot(x1, w_ref[...], preferred_element_type=jnp.float32) + b_ref[...]
    mu = jnp.mean(t, axis=1, keepdims=True)
    var = jnp.mean((t - mu) ** 2, axis=1, keepdims=True)
    xn = (t - mu) * lax.rsqrt(var + 1e-5) * g_ref[...] + gb_ref[...]
    out_ref[...] = nf_ref[...] + xn * jax.nn.sigmoid(xn)


def _x_update(sums, axu, nf, wxg, bxg, gn, bnb):
    nspec = pl.BlockSpec((BN, D), lambda i: (i, 0))
    return pl.pallas_call(
        _x_body,
        grid=(N // BN,),
        in_specs=[pl.BlockSpec((BN, D), lambda i: (i, 1)),
                  pl.BlockSpec((BN, D), lambda i: (i, 0)),
                  nspec, nspec,
                  pl.BlockSpec((D, D), lambda i: (0, 0)),
                  pl.BlockSpec((1, D), lambda i: (0, 0)),
                  pl.BlockSpec((1, D), lambda i: (0, 0)),
                  pl.BlockSpec((1, D), lambda i: (0, 0))],
        out_specs=nspec,
        out_shape=jax.ShapeDtypeStruct((N, D), jnp.float32),
    )(sums, sums, axu, nf, wxg, bxg, gn, bnb)


_MESH = None
_SC_PARAMS = pltpu.CompilerParams(needs_layout_passes=False)


def _mesh():
    global _MESH
    if _MESH is None:
        _MESH = plsc.VectorSubcoreMesh(core_axis_name="c", subcore_axis_name="s")
    return _MESH


def _zero16():
    return jnp.zeros((16,), jnp.float32)


def _splat_scalar(vec16):
    return lax.reduce_max(vec16, axes=(0,))


def _denom_body(src_hbm, dst_hbm, as_hbm, ad_hbm, dp_out,
                atts_v, attd_v, src_v, dst_v, dstn_v, es_v, ed_v, zbuf, acc):
    c = lax.axis_index("c")
    s = lax.axis_index("s")
    w = c * 16 + s
    pltpu.sync_copy(as_hbm, atts_v)
    pltpu.sync_copy(ad_hbm, attd_v)
    for j in range(ZB // 16):
        zbuf[pl.ds(j * 16, 16)] = _zero16()
    pltpu.sync_copy(zbuf, acc.at[pl.ds(s * ZB, ZB)])

    @pl.when(s == 0)
    def _():
        pltpu.sync_copy(zbuf.at[pl.ds(0, 2 * N - 16 * ZB)],
                        acc.at[pl.ds(16 * ZB, 2 * N - 16 * ZB)])

    plsc.subcore_barrier()

    def chunk(i, carry):
        base = w * EW + i * CHB
        pltpu.sync_copy(src_hbm.at[pl.ds(base, CHB)], src_v)
        pltpu.sync_copy(dst_hbm.at[pl.ds(base, CHB)], dst_v)
        for j in range(CHB // 16):
            sl = pl.ds(j * 16, 16)
            sv = src_v[sl]
            dv = dst_v[sl]
            es_v[sl] = jnp.exp(plsc.load_gather(atts_v, [sv]))
            ed_v[sl] = jnp.exp(plsc.load_gather(attd_v, [sv]))
            dstn_v[sl] = dv + N
        pltpu.sync_copy(es_v, acc.at[dst_v], add=True)
        pltpu.sync_copy(ed_v, acc.at[dstn_v], add=True)
        return carry

    lax.fori_loop(0, EW // CHB, chunk, 0)
    plsc.subcore_barrier()
    pltpu.sync_copy(acc.at[pl.ds(s * ZB, ZB)], zbuf)
    pltpu.sync_copy(zbuf, dp_out.at[pl.ds(c * 2 * N + s * ZB, ZB)])

    @pl.when(s == 0)
    def _():
        pltpu.sync_copy(acc.at[pl.ds(16 * ZB, 2 * N - 16 * ZB)],
                        zbuf.at[pl.ds(0, 2 * N - 16 * ZB)])
        pltpu.sync_copy(zbuf.at[pl.ds(0, 2 * N - 16 * ZB)],
                        dp_out.at[pl.ds(c * 2 * N + 16 * ZB, 2 * N - 16 * ZB)])


def _sc_denoms(src, dst, as_sh, ad_sh):
    f = functools.partial(
        pl.kernel,
        out_type=jax.ShapeDtypeStruct((4 * N,), jnp.float32),
        mesh=_mesh(),
        scratch_types=[
            pltpu.VMEM((N,), jnp.float32),
            pltpu.VMEM((N,), jnp.float32),
            pltpu.VMEM((CHB,), jnp.int32),
            pltpu.VMEM((CHB,), jnp.int32),
            pltpu.VMEM((CHB,), jnp.int32),
            pltpu.VMEM((CHB,), jnp.float32),
            pltpu.VMEM((CHB,), jnp.float32),
            pltpu.VMEM((ZB,), jnp.float32),
            pltpu.VMEM_SHARED((2 * N,), jnp.float32),
        ],
        compiler_params=_SC_PARAMS,
    )(_denom_body)
    return f(src, dst, as_sh, ad_sh)


def _rec_body(src_hbm, dst_hbm, as_hbm, ad_hbm, dsum_hbm, start_hbm, bi_hbm,
              ras_out, rad_out, rsrc_out, reid_out, rdl_out,
              atts_v, attd_v, ds_v, dd_v, src_v, dst_v, pos_v, zf_v,
              asc_v, adc_v, eid_c, dl_c, binfo_v, start_v, runs_v, z16, zi16):
    c = lax.axis_index("c")
    s = lax.axis_index("s")
    w = c * 16 + s
    pltpu.sync_copy(as_hbm, atts_v)
    pltpu.sync_copy(ad_hbm, attd_v)
    pltpu.sync_copy(dsum_hbm.at[pl.ds(0, N)], ds_v)
    pltpu.sync_copy(dsum_hbm.at[pl.ds(N, N)], dd_v)
    pltpu.sync_copy(start_hbm.at[pl.ds(w * NBIN * 16, NBIN * 16)], start_v)
    pltpu.sync_copy(bi_hbm, binfo_v)
    z16[pl.ds(0, 16)] = _zero16()
    zi16[pl.ds(0, 16)] = jnp.zeros((16,), jnp.int32)

    @pl.when(w == 0)
    def _():
        for b in range(NBIN):
            bb = _splat_scalar(binfo_v[pl.ds(b * 16, 16)])
            bnext = _splat_scalar(binfo_v[pl.ds((b + 1) * 16, 16)])
            cntb = _splat_scalar(binfo_v[pl.ds((33 + b) * 16, 16)])
            idx = bb + cntb + lax.iota(jnp.int32, 16)
            zf_v[0, pl.ds(0, 16)] = jnp.where(idx < bnext, idx, RTRASH)
            pltpu.sync_copy(zi16, rsrc_out.at[zf_v.at[0]])
            pltpu.sync_copy(zi16, reid_out.at[zf_v.at[0]])
        total = _splat_scalar(binfo_v[pl.ds(32 * 16, 16)])
        for t in range((RECN - E + 15) // 16 + 1):
            idx = total + t * 16 + lax.iota(jnp.int32, 16)
            zf_v[0, pl.ds(0, 16)] = jnp.where(idx < RECN, idx, RTRASH)
            pltpu.sync_copy(zi16, rsrc_out.at[zf_v.at[0]])
            pltpu.sync_copy(zi16, reid_out.at[zf_v.at[0]])

    runs0 = tuple(start_v[pl.ds(b * 16, 16)] for b in range(NBIN))

    def chunk(i, runs):
        base = w * EW + i * CHC
        pltpu.sync_copy(src_hbm.at[pl.ds(base, CHC)], src_v)
        pltpu.sync_copy(dst_hbm.at[pl.ds(base, CHC)], dst_v)
        runs = list(runs)
        for j in range(CHC // 16):
            sl = pl.ds(j * 16, 16)
            sv = src_v[sl]
            dv = dst_v[sl]
            es = jnp.exp(plsc.load_gather(atts_v, [sv]))
            ed = jnp.exp(plsc.load_gather(attd_v, [sv]))
            asc_v[sl] = es / plsc.load_gather(ds_v, [dv])
            adc_v[sl] = ed / plsc.load_gather(dd_v, [dv])
            bins = ((dv.astype(jnp.float32) + 0.5) *
                    (1.0 / BINW)).astype(jnp.int32)
            dl_c[sl] = dv - bins * BINW
            eid_c[sl] = base + j * 16 + lax.iota(jnp.int32, 16)
            pos = jnp.zeros((16,), jnp.int32)
            for b in range(NBIN):
                mb = bins == b
                cum = plsc.cumsum(mb.astype(jnp.int32))
                nbv = lax.reduce_max(cum, axes=(0,))
                rbv = runs[b]
                pos = jnp.where(mb, rbv + cum - 1, pos)
                runs[b] = rbv + nbv
            pos_v[0, sl] = pos
        pltpu.sync_copy(asc_v, ras_out.at[pos_v.at[0]])
        pltpu.sync_copy(adc_v, rad_out.at[pos_v.at[0]])
        pltpu.sync_copy(src_v, rsrc_out.at[pos_v.at[0]])
        pltpu.sync_copy(eid_c, reid_out.at[pos_v.at[0]])
        pltpu.sync_copy(dl_c, rdl_out.at[pos_v.at[0]])
        return tuple(runs)

    lax.fori_loop(0, EW // CHC, chunk, runs0)


def _sc_records(src, dst, as_sh, ad_sh, dsum, start, binfo):
    f = functools.partial(
        pl.kernel,
        out_type=[jax.ShapeDtypeStruct((RECN + 8,), jnp.float32),
                  jax.ShapeDtypeStruct((RECN + 8,), jnp.float32),
                  jax.ShapeDtypeStruct((RECN + 8,), jnp.int32),
                  jax.ShapeDtypeStruct((RECN + 8,), jnp.int32),
                  jax.ShapeDtypeStruct((RECN + 8,), jnp.int32)],
        mesh=_mesh(),
        scratch_types=[
            pltpu.VMEM((N,), jnp.float32),
            pltpu.VMEM((N,), jnp.float32),
            pltpu.VMEM((N,), jnp.float32),
            pltpu.VMEM((N,), jnp.float32),
            pltpu.VMEM((CHC,), jnp.int32),
            pltpu.VMEM((CHC,), jnp.int32),
            pltpu.VMEM((2, CHC), jnp.int32),
            pltpu.VMEM((2, 16), jnp.int32),
            pltpu.VMEM((CHC,), jnp.float32),
            pltpu.VMEM((CHC,), jnp.float32),
            pltpu.VMEM((CHC,), jnp.int32),
            pltpu.VMEM((CHC,), jnp.int32),
            pltpu.VMEM((BI_LEN,), jnp.int32),
            pltpu.VMEM((NBIN * 16,), jnp.int32),
            pltpu.VMEM((NBIN, 16), jnp.int32),
            pltpu.VMEM((16,), jnp.float32),
            pltpu.VMEM((16,), jnp.int32),
        ],
        compiler_params=_SC_PARAMS,
    )(_rec_body)
    return f(src, dst, as_sh, ad_sh, dsum, start, binfo)


def _edge_body(ras_hbm, rad_hbm, rsrc_hbm, reid_hbm, rdl_hbm, bi_hbm,
               t_hbm, ef_hbm, m_out, sums_out,
               asc_v, adc_v, src_v, eid_v, dl_v, midx_v, nidx_v, binfo_v,
               trow_v, ef_v, acc_v, sem):
    c = lax.axis_index("c")
    s = lax.axis_index("s")
    w = c * 16 + s
    pltpu.sync_copy(bi_hbm, binfo_v)
    bb = pl.multiple_of(_splat_scalar(binfo_v[pl.ds(w * 16, 16)]), 8)
    cnt = _splat_scalar(binfo_v[pl.ds((33 + w) * 16, 16)])

    def zrow(k, carry):
        for j in range(16):
            acc_v[k, pl.ds(j * 16, 16)] = _zero16()
        return carry

    lax.fori_loop(0, ACC_ROWS, zrow, 0)

    def chunk(i, carry):
        rb = bb + i * CHD
        pltpu.sync_copy(ras_hbm.at[pl.ds(rb, CHD)], asc_v)
        pltpu.sync_copy(rad_hbm.at[pl.ds(rb, CHD)], adc_v)
        pltpu.sync_copy(rsrc_hbm.at[pl.ds(rb, CHD)], src_v)
        pltpu.sync_copy(reid_hbm.at[pl.ds(rb, CHD)], eid_v)
        pltpu.sync_copy(rdl_hbm.at[pl.ds(rb, CHD)], dl_v)
        nv = jnp.minimum(CHD, cnt - i * CHD)
        for j in range(CHD // 16):
            sl = pl.ds(j * 16, 16)
            kvec = jnp.full((16,), j * 16, jnp.int32) + lax.iota(jnp.int32, 16)
            midx_v[0, sl] = jnp.where(kvec < nv, eid_v[sl], E)
        pltpu.async_copy(t_hbm.at[src_v], trow_v, sem).wait()
        pltpu.async_copy(ef_hbm.at[eid_v], ef_v, sem).wait()

        def edge(k, carry2):
            k16 = jnp.full((16,), 0, jnp.int32) + k
            aq = plsc.load_gather(asc_v, [k16])
            bq = plsc.load_gather(adc_v, [k16])
            dl = _splat_scalar(plsc.load_gather(dl_v, [k16]))
            for j2 in range(8):
                d0 = pl.ds(j2 * 16, 16)
                d1 = pl.ds(D + j2 * 16, 16)
                d2 = pl.ds(2 * D + j2 * 16, 16)
                m = ef_v[k, d0] + aq * trow_v[k, d0] + bq * trow_v[k, d1]
                ef_v[k, d0] = m
                sg = 1.0 / (1.0 + jnp.exp(-m))
                acc_v[dl, d0] = acc_v[dl, d0] + sg
                acc_v[dl, d1] = acc_v[dl, d1] + trow_v[k, d2] * sg
            return carry2

        lax.fori_loop(0, nv, edge, 0)
        pltpu.sync_copy(ef_v, m_out.at[midx_v.at[0]])
        return carry

    nchunks = (cnt + CHD - 1) // CHD
    lax.fori_loop(0, nchunks, chunk, 0)

    nbase = w * BINW
    for q in range(4):
        for j in range(5):
            r = jnp.full((16,), q * 80 + j * 16, jnp.int32) + lax.iota(jnp.int32, 16)
            node = nbase + r
            nidx_v[q, pl.ds(j * 16, 16)] = jnp.where((r < BINW) & (node < N),
                                                     node, N)
    for q in range(4):
        pltpu.sync_copy(acc_v.at[pl.ds(q * 80, 80)], sums_out.at[nidx_v.at[q]])


def _sc_edges(ras, rad, rsrc, reid, rdl, binfo, t_tab, ef):
    f = functools.partial(
        pl.kernel,
        out_type=[jax.ShapeDtypeStruct((E + 8, D), jnp.float32),
                  jax.ShapeDtypeStruct((N + 8, 2 * D), jnp.float32)],
        mesh=_mesh(),
        scratch_types=[
            pltpu.VMEM((CHD,), jnp.float32),
            pltpu.VMEM((CHD,), jnp.float32),
            pltpu.VMEM((CHD,), jnp.int32),
            pltpu.VMEM((CHD,), jnp.int32),
            pltpu.VMEM((CHD,), jnp.int32),
            pltpu.VMEM((2, CHD), jnp.int32),
            pltpu.VMEM((4, 80), jnp.int32),
            pltpu.VMEM((BI_LEN,), jnp.int32),
            pltpu.VMEM((CHD, 3 * D), jnp.float32),
            pltpu.VMEM((CHD, D), jnp.float32),
            pltpu.VMEM((ACC_ROWS, 2 * D), jnp.float32),
            pltpu.SemaphoreType.DMA,
        ],
        compiler_params=_SC_PARAMS,
    )(_edge_body)
    return f(ras, rad, rsrc, reid, rdl, binfo, t_tab, ef)


def kernel(node_feats, edge_feats, edge_index, Wsg, bsg, Wdg, bdg, Weg, beg,
           Wxg, bxg, Wmg, bmg, att_src, att_dst, Wsu, bsu, Wdu, bdu,
           gn, bnb, ge, beb):
    src = edge_index[0].astype(jnp.int32)
    dst = edge_index[1].astype(jnp.int32)
    vs = att_src.reshape(D, 1)
    vd = att_dst.reshape(D, 1)

    t_tab, atts, attd, axu = _node_precompute(
        node_feats, Wsg, bsg.reshape(1, D), Wdg, bdg.reshape(1, D),
        Wdu, bdu.reshape(1, D), Wsu, bsu.reshape(1, D), vs, vd)
    as_sh, ad_sh = _shift_att(atts, attd)
    as_sh = as_sh.reshape(N)
    ad_sh = ad_sh.reshape(N)

    ef_lin = _edge_matmul(edge_feats, Weg, beg.reshape(1, D))
    cnt = _bin_hist(dst.reshape(32, 1, EW)).reshape(32, NBIN)

    dp = _sc_denoms(src, dst, as_sh, ad_sh).reshape(2, 2 * N)
    dsum = _reduce_denoms(dp)
    start, binfo = _bin_plan(cnt)
    ras, rad, rsrc, reid, rdl = _sc_records(
        src, dst, as_sh, ad_sh, dsum.reshape(2 * N), start, binfo)

    m_full, sums = _sc_edges(ras, rad, rsrc, reid, rdl, binfo, t_tab, ef_lin)

    y = _y_update(m_full, edge_feats, Wmg, bmg.reshape(1, D),
                  ge.reshape(1, D), beb.reshape(1, D))
    x = _x_update(sums, axu, node_feats, Wxg, bxg.reshape(1, D),
                  gn.reshape(1, D), bnb.reshape(1, D))
    return (x, y)
